# baseline (speedup 1.0000x reference)
import functools

import jax
import jax.numpy as jnp
from jax import lax
from jax.experimental import pallas as pl
from jax.experimental.pallas import tpu as pltpu
from jax.experimental.pallas import tpu_sc as plsc

CHUNK = 64
LANES = 16


@functools.lru_cache(maxsize=None)
def _make_lookup(B, V, D):
    info = plsc.get_sparse_core_info()
    NC, NS = info.num_cores, info.num_subcores
    NW = NC * NS
    assert B % (NW * CHUNK) == 0
    b_per_w = B // NW
    n = b_per_w // CHUNK
    assert n % 2 == 0 and n >= 4

    mesh = plsc.VectorSubcoreMesh(core_axis_name="c", subcore_axis_name="s")

    @functools.partial(
        pl.kernel,
        mesh=mesh,
        out_type=jax.ShapeDtypeStruct((B, D), jnp.float32),
        scratch_types=[
            pltpu.VMEM((V, D), jnp.float32),
            pltpu.VMEM((n, CHUNK), jnp.int32),
            pltpu.VMEM((CHUNK, D), jnp.float32),
            pltpu.VMEM((CHUNK, D), jnp.float32),
            pltpu.SemaphoreType.DMA,
            pltpu.SemaphoreType.DMA,
        ],
    )
    def lookup(idx_hbm, table_hbm, out_hbm, table_v, idx_v, buf0, buf1,
               osem0, osem1):
        bufs = (buf0, buf1)
        osems = (osem0, osem1)

        wid = lax.axis_index("s") * NC + lax.axis_index("c")
        base = wid * b_per_w
        pltpu.sync_copy(table_hbm, table_v)
        pltpu.sync_copy(idx_hbm.at[wid], idx_v)

        def compute(g, b):
            def quarter(k, carry):
                iv = idx_v[g, pl.ds(k * LANES, LANES)]
                for l in range(LANES):
                    i = iv[l]
                    r = k * LANES + l
                    for c in range(D // LANES):
                        sl = pl.ds(c * LANES, LANES)
                        bufs[b][r, sl] = table_v[i, sl]
                return carry

            lax.fori_loop(0, CHUNK // LANES, quarter, 0)

        def start_write(g, b):
            pltpu.async_copy(
                bufs[b], out_hbm.at[pl.ds(base + g * CHUNK, CHUNK)], osems[b]
            )

        def wait_write(g, b):
            pltpu.make_async_copy(
                bufs[b], out_hbm.at[pl.ds(base + g * CHUNK, CHUNK)], osems[b]
            ).wait()

        for g in (0, 1):
            compute(g, g)
            start_write(g, g)

        def pair(q, carry):
            for j in range(2):
                g = 2 * q + j
                wait_write(g - 2, j)
                compute(g, j)
                start_write(g, j)
            return carry

        lax.fori_loop(1, n // 2, pair, 0)

        wait_write(n - 2, 0)
        wait_write(n - 1, 1)

    return lookup


def kernel(input_ids, embed_weight):
    B = input_ids.shape[0] * input_ids.shape[1]
    V, D = embed_weight.shape
    info = plsc.get_sparse_core_info()
    NW = info.num_cores * info.num_subcores
    idx = input_ids.reshape(NW, (B // NW) // CHUNK, CHUNK).astype(jnp.int32)
    out = _make_lookup(B, V, D)(idx, embed_weight)
    return out.reshape(input_ids.shape[0], input_ids.shape[1], D)
